# Initial kernel scaffold; baseline (speedup 1.0000x reference)
import functools

import jax
import jax.numpy as jnp
from jax import lax
from jax.experimental import pallas as pl
from jax.experimental.pallas import tpu as pltpu
from jax.experimental.pallas import tpu_sc as plsc

_N = 10000
_D = 128
_E = 320000
_K = 2

_NS = 16
_NPAD = 10240
_NT = _NPAD // _NS
_ROWS = 160
_EC = _ROWS * 128
_EPAD = _NS * _EC
_DUMMY = _NPAD - 1
_VL = 16


def _matvec(xp, W):
    def body(x_ref, w_ref, o_ref):
        o_ref[...] = jnp.sum(x_ref[...] * w_ref[...], axis=1, keepdims=True)

    out = pl.pallas_call(
        body,
        grid=(16,),
        in_specs=[
            pl.BlockSpec((_NPAD // 16, _D), lambda i: (i, 0)),
            pl.BlockSpec((1, _D), lambda i: (0, 0)),
        ],
        out_specs=pl.BlockSpec((_NPAD // 16, 1), lambda i: (i, 0)),
        out_shape=jax.ShapeDtypeStruct((_NPAD, 1), jnp.float32),
    )(xp, W)
    return out.reshape(_NPAD)


def _sc_propagate(srcp, dstp, y0):
    mesh = plsc.VectorSubcoreMesh(
        core_axis_name="c", subcore_axis_name="s", num_cores=1
    )

    @functools.partial(
        pl.kernel,
        out_type=jax.ShapeDtypeStruct((_NPAD,), jnp.float32),
        mesh=mesh,
        scratch_types=[
            pltpu.VMEM((_ROWS, 128), jnp.int32),
            pltpu.VMEM((_ROWS, 128), jnp.int32),
            pltpu.VMEM((_ROWS, 128), jnp.float32),
            pltpu.VMEM((_NT,), jnp.float32),
            pltpu.VMEM((_NT,), jnp.float32),
            pltpu.VMEM((_NT,), jnp.float32),
            pltpu.VMEM((_NT,), jnp.float32),
            pltpu.VMEM((_NT,), jnp.float32),
            pltpu.VMEM_SHARED((_NPAD,), jnp.float32),
            pltpu.VMEM_SHARED((_NPAD,), jnp.float32),
        ],
    )
    def run(src_hbm, dst_hbm, y0_hbm, out_hbm,
            src_t, dst_t, vals, y_t, z_t, dinv_t, tmp_t, zer_t,
            acc_sh, z_sh):
        sid = lax.axis_index("s")
        base = sid * _NT

        pltpu.sync_copy(src_hbm.at[sid], src_t)
        pltpu.sync_copy(dst_hbm.at[sid], dst_t)
        pltpu.sync_copy(y0_hbm.at[pl.ds(base, _NT)], y_t)

        def fill_zeros(i, _):
            zer_t[pl.ds(i * _VL, _VL)] = jnp.zeros((_VL,), jnp.float32)
            return 0

        lax.fori_loop(0, _NT // _VL, fill_zeros, 0)

        ones = jnp.ones((_VL,), jnp.float32)

        def fill_ones(j, _):
            for l in range(128 // _VL):
                vals[j, pl.ds(l * _VL, _VL)] = ones
            return 0

        lax.fori_loop(0, _ROWS, fill_ones, 0)

        pltpu.sync_copy(zer_t, acc_sh.at[pl.ds(base, _NT)])
        plsc.subcore_barrier()

        pltpu.sync_copy(vals, acc_sh.at[dst_t], add=True)
        plsc.subcore_barrier()

        pltpu.sync_copy(acc_sh.at[pl.ds(base, _NT)], tmp_t)

        def mk_dinv(i, _):
            s = pl.ds(i * _VL, _VL)
            dg = tmp_t[s] + 1.0
            bits = plsc.bitcast(dg, jnp.int32)
            bits = 0x5F3759DF - lax.shift_right_arithmetic(bits, 1)
            r = plsc.bitcast(bits, jnp.float32)
            for _ in range(3):
                r = r * (1.5 - 0.5 * dg * r * r)
            dinv_t[s] = r
            return 0

        lax.fori_loop(0, _NT // _VL, mk_dinv, 0)
        pltpu.sync_copy(zer_t, acc_sh.at[pl.ds(base, _NT)])

        for k in range(_K):
            def mk_z(i, _):
                s = pl.ds(i * _VL, _VL)
                z_t[s] = dinv_t[s] * y_t[s]
                return 0

            lax.fori_loop(0, _NT // _VL, mk_z, 0)
            pltpu.sync_copy(z_t, z_sh.at[pl.ds(base, _NT)])
            plsc.subcore_barrier()

            pltpu.sync_copy(z_sh.at[src_t], vals)
            pltpu.sync_copy(vals, acc_sh.at[dst_t], add=True)
            plsc.subcore_barrier()

            pltpu.sync_copy(acc_sh.at[pl.ds(base, _NT)], tmp_t)
            if k < _K - 1:
                pltpu.sync_copy(zer_t, acc_sh.at[pl.ds(base, _NT)])

            def upd_y(i, _):
                s = pl.ds(i * _VL, _VL)
                y_t[s] = dinv_t[s] * (tmp_t[s] + z_t[s])
                return 0

            lax.fori_loop(0, _NT // _VL, upd_y, 0)

        pltpu.sync_copy(y_t, out_hbm.at[pl.ds(base, _NT)])

    return run(srcp, dstp, y0)


def kernel(x, edge_index, W, b):
    src = edge_index[0].astype(jnp.int32)
    dst = edge_index[1].astype(jnp.int32)
    pad = jnp.full((_EPAD - _E,), _DUMMY, jnp.int32)
    srcp = jnp.concatenate([src, pad]).reshape(_NS, _ROWS, 128)
    dstp = jnp.concatenate([dst, pad]).reshape(_NS, _ROWS, 128)
    xp = jnp.pad(x, ((0, _NPAD - _N), (0, 0)))

    y0 = _matvec(xp, W)
    out = _sc_propagate(srcp, dstp, y0)
    return out[:_N] + b[0]


# speedup vs baseline: 75.3231x; 75.3231x over previous
import functools

import jax
import jax.numpy as jnp
from jax import lax
from jax.experimental import pallas as pl
from jax.experimental.pallas import tpu as pltpu
from jax.experimental.pallas import tpu_sc as plsc

_N = 10000
_D = 128
_E = 320000
_K = 2

_NS = 16
_NPAD = 10240
_NT = _NPAD // _NS
_ROWS = 160
_EC = _ROWS * 128
_EPAD = _NS * _EC
_DUMMY = _NPAD - 1
_VL = 16


def _matvec(xp, W):
    def body(x_ref, w_ref, o_ref):
        o_ref[...] = jnp.sum(x_ref[...] * w_ref[...], axis=1, keepdims=True)

    out = pl.pallas_call(
        body,
        grid=(16,),
        in_specs=[
            pl.BlockSpec((_NPAD // 16, _D), lambda i: (i, 0)),
            pl.BlockSpec((1, _D), lambda i: (0, 0)),
        ],
        out_specs=pl.BlockSpec((_NPAD // 16, 1), lambda i: (i, 0)),
        out_shape=jax.ShapeDtypeStruct((_NPAD, 1), jnp.float32),
    )(xp, W)
    return out.reshape(_NPAD)


def _sc_propagate(srcp, dstp, y0):
    mesh = plsc.VectorSubcoreMesh(
        core_axis_name="c", subcore_axis_name="s", num_cores=1
    )

    @functools.partial(
        pl.kernel,
        out_type=jax.ShapeDtypeStruct((_NPAD,), jnp.float32),
        mesh=mesh,
        scratch_types=[
            pltpu.VMEM((_EC,), jnp.int32),
            pltpu.VMEM((_EC,), jnp.int32),
            pltpu.VMEM((_EC,), jnp.float32),
            pltpu.VMEM((_NT,), jnp.float32),
            pltpu.VMEM((_NT,), jnp.float32),
            pltpu.VMEM((_NT,), jnp.float32),
            pltpu.VMEM((_NT,), jnp.float32),
            pltpu.VMEM((_NT,), jnp.float32),
            pltpu.VMEM_SHARED((_NPAD,), jnp.float32),
            pltpu.VMEM_SHARED((_NPAD,), jnp.float32),
        ],
    )
    def run(src_hbm, dst_hbm, y0_hbm, out_hbm,
            src_t, dst_t, vals, y_t, z_t, dinv_t, tmp_t, zer_t,
            acc_sh, z_sh):
        sid = lax.axis_index("s")
        base = sid * _NT

        pltpu.sync_copy(src_hbm.at[sid], src_t)
        pltpu.sync_copy(dst_hbm.at[sid], dst_t)
        pltpu.sync_copy(y0_hbm.at[pl.ds(base, _NT)], y_t)

        def fill_zeros(i, _):
            zer_t[pl.ds(i * _VL, _VL)] = jnp.zeros((_VL,), jnp.float32)
            return 0

        lax.fori_loop(0, _NT // _VL, fill_zeros, 0)

        ones = jnp.ones((_VL,), jnp.float32)

        def fill_ones(j, _):
            vals[pl.ds(j * _VL, _VL)] = ones
            return 0

        lax.fori_loop(0, _EC // _VL, fill_ones, 0)

        pltpu.sync_copy(zer_t, acc_sh.at[pl.ds(base, _NT)])
        plsc.subcore_barrier()

        pltpu.sync_copy(vals, acc_sh.at[dst_t], add=True)
        plsc.subcore_barrier()

        pltpu.sync_copy(acc_sh.at[pl.ds(base, _NT)], tmp_t)

        def mk_dinv(i, _):
            s = pl.ds(i * _VL, _VL)
            dg = tmp_t[s] + 1.0
            bits = lax.bitcast_convert_type(dg, jnp.int32)
            bits = 0x5F3759DF - lax.shift_right_arithmetic(bits, 1)
            r = lax.bitcast_convert_type(bits, jnp.float32)
            for _ in range(3):
                r = r * (1.5 - 0.5 * dg * r * r)
            dinv_t[s] = r
            return 0

        lax.fori_loop(0, _NT // _VL, mk_dinv, 0)
        pltpu.sync_copy(zer_t, acc_sh.at[pl.ds(base, _NT)])

        for k in range(_K):
            def mk_z(i, _):
                s = pl.ds(i * _VL, _VL)
                z_t[s] = dinv_t[s] * y_t[s]
                return 0

            lax.fori_loop(0, _NT // _VL, mk_z, 0)
            pltpu.sync_copy(z_t, z_sh.at[pl.ds(base, _NT)])
            plsc.subcore_barrier()

            pltpu.sync_copy(z_sh.at[src_t], vals)
            pltpu.sync_copy(vals, acc_sh.at[dst_t], add=True)
            plsc.subcore_barrier()

            pltpu.sync_copy(acc_sh.at[pl.ds(base, _NT)], tmp_t)
            if k < _K - 1:
                pltpu.sync_copy(zer_t, acc_sh.at[pl.ds(base, _NT)])

            def upd_y(i, _):
                s = pl.ds(i * _VL, _VL)
                y_t[s] = dinv_t[s] * (tmp_t[s] + z_t[s])
                return 0

            lax.fori_loop(0, _NT // _VL, upd_y, 0)

        pltpu.sync_copy(y_t, out_hbm.at[pl.ds(base, _NT)])

    return run(srcp, dstp, y0)


def kernel(x, edge_index, W, b):
    src = edge_index[0].astype(jnp.int32)
    dst = edge_index[1].astype(jnp.int32)
    pad = jnp.full((_EPAD - _E,), _DUMMY, jnp.int32)
    srcp = jnp.concatenate([src, pad]).reshape(_NS, _EC)
    dstp = jnp.concatenate([dst, pad]).reshape(_NS, _EC)
    xp = jnp.pad(x, ((0, _NPAD - _N), (0, 0)))

    y0 = _matvec(xp, W)
    out = _sc_propagate(srcp, dstp, y0)
    return out[:_N] + b[0]


# speedup vs baseline: 114.8744x; 1.5251x over previous
import functools

import jax
import jax.numpy as jnp
from jax import lax
from jax.experimental import pallas as pl
from jax.experimental.pallas import tpu as pltpu
from jax.experimental.pallas import tpu_sc as plsc

_N = 10000
_D = 128
_E = 320000
_K = 2

_NS = 16
_NPAD = 10240
_NT = _NPAD // _NS
_EC = _E // _NS
_VL = 16


def _matvec(x, W):
    def body(x_ref, w_ref, o_ref):
        o_ref[...] = jnp.sum(x_ref[...] * w_ref[...], axis=1, keepdims=True)

    return pl.pallas_call(
        body,
        grid=(10,),
        in_specs=[
            pl.BlockSpec((_N // 10, _D), lambda i: (i, 0)),
            pl.BlockSpec((1, _D), lambda i: (0, 0)),
        ],
        out_specs=pl.BlockSpec((_N // 10, 1), lambda i: (i, 0)),
        out_shape=jax.ShapeDtypeStruct((_N, 1), jnp.float32),
    )(x, W)


def _sc_propagate(srcv, dstv, y0):
    mesh = plsc.VectorSubcoreMesh(
        core_axis_name="c", subcore_axis_name="s", num_cores=1
    )

    @functools.partial(
        pl.kernel,
        out_type=jax.ShapeDtypeStruct((_NPAD,), jnp.float32),
        mesh=mesh,
        scratch_types=[
            pltpu.VMEM((_EC,), jnp.int32),
            pltpu.VMEM((_EC,), jnp.int32),
            pltpu.VMEM((_EC,), jnp.float32),
            pltpu.VMEM((_NT,), jnp.float32),
            pltpu.VMEM((_NT,), jnp.float32),
            pltpu.VMEM((_NT,), jnp.float32),
            pltpu.VMEM((_NT,), jnp.float32),
            pltpu.VMEM((_NT,), jnp.float32),
            pltpu.VMEM_SHARED((_NPAD,), jnp.float32),
            pltpu.VMEM_SHARED((_NPAD,), jnp.float32),
        ],
    )
    def run(src_hbm, dst_hbm, y0_hbm, out_hbm,
            src_t, dst_t, vals, y_t, z_t, dinv_t, tmp_t, zer_t,
            acc_sh, z_sh):
        sid = lax.axis_index("s")
        base = sid * _NT
        ebase = sid * _EC

        pltpu.sync_copy(src_hbm.at[pl.ds(ebase, _EC)], src_t)
        pltpu.sync_copy(dst_hbm.at[pl.ds(ebase, _EC)], dst_t)
        pltpu.sync_copy(y0_hbm.at[pl.ds(base, _NT)], y_t)

        def fill_zeros(i, _):
            zer_t[pl.ds(i * _VL, _VL)] = jnp.zeros((_VL,), jnp.float32)
            return 0

        lax.fori_loop(0, _NT // _VL, fill_zeros, 0)

        ones = jnp.ones((_VL,), jnp.float32)

        def fill_ones(j, _):
            vals[pl.ds(j * _VL, _VL)] = ones
            return 0

        lax.fori_loop(0, _EC // _VL, fill_ones, 0)

        pltpu.sync_copy(zer_t, acc_sh.at[pl.ds(base, _NT)])
        plsc.subcore_barrier()

        pltpu.sync_copy(vals, acc_sh.at[dst_t], add=True)
        plsc.subcore_barrier()

        pltpu.sync_copy(acc_sh.at[pl.ds(base, _NT)], tmp_t)

        def mk_dinv(i, _):
            s = pl.ds(i * _VL, _VL)
            dg = tmp_t[s] + 1.0
            bits = lax.bitcast_convert_type(dg, jnp.int32)
            bits = 0x5F3759DF - lax.shift_right_arithmetic(bits, 1)
            r = lax.bitcast_convert_type(bits, jnp.float32)
            for _ in range(3):
                r = r * (1.5 - 0.5 * dg * r * r)
            dinv_t[s] = r
            return 0

        lax.fori_loop(0, _NT // _VL, mk_dinv, 0)
        pltpu.sync_copy(zer_t, acc_sh.at[pl.ds(base, _NT)])

        for k in range(_K):
            def mk_z(i, _):
                s = pl.ds(i * _VL, _VL)
                z_t[s] = dinv_t[s] * y_t[s]
                return 0

            lax.fori_loop(0, _NT // _VL, mk_z, 0)
            pltpu.sync_copy(z_t, z_sh.at[pl.ds(base, _NT)])
            plsc.subcore_barrier()

            pltpu.sync_copy(z_sh.at[src_t], vals)
            pltpu.sync_copy(vals, acc_sh.at[dst_t], add=True)
            plsc.subcore_barrier()

            pltpu.sync_copy(acc_sh.at[pl.ds(base, _NT)], tmp_t)
            if k < _K - 1:
                pltpu.sync_copy(zer_t, acc_sh.at[pl.ds(base, _NT)])

            def upd_y(i, _):
                s = pl.ds(i * _VL, _VL)
                y_t[s] = dinv_t[s] * (tmp_t[s] + z_t[s])
                return 0

            lax.fori_loop(0, _NT // _VL, upd_y, 0)

        pltpu.sync_copy(y_t, out_hbm.at[pl.ds(base, _NT)])

    return run(srcv, dstv, y0)


def kernel(x, edge_index, W, b):
    ei = edge_index.astype(jnp.int32)
    y0 = _matvec(x, W).reshape(_N)
    y0p = jnp.pad(y0, (0, _NPAD - _N))
    out = _sc_propagate(ei[0], ei[1], y0p)
    return out[:_N] + b[0]


# speedup vs baseline: 121.5747x; 1.0583x over previous
import functools

import jax
import jax.numpy as jnp
from jax import lax
from jax.experimental import pallas as pl
from jax.experimental.pallas import tpu as pltpu
from jax.experimental.pallas import tpu_sc as plsc

_N = 10000
_D = 128
_E = 320000
_K = 2

_NS = 16
_NPAD = 10240
_NT = _NPAD // _NS
_EC = _E // _NS
_VL = 16


def _matvec(x, W):
    def body(x_ref, w_ref, o_ref):
        o_ref[...] = jnp.sum(x_ref[...] * w_ref[...], axis=1, keepdims=True)

    return pl.pallas_call(
        body,
        grid=(10,),
        in_specs=[
            pl.BlockSpec((_N // 10, _D), lambda i: (i, 0)),
            pl.BlockSpec((1, _D), lambda i: (0, 0)),
        ],
        out_specs=pl.BlockSpec((_N // 10, 1), lambda i: (i, 0)),
        out_shape=jax.ShapeDtypeStruct((_N, 1), jnp.float32),
    )(x, W)


def _sc_propagate(srcv, dstv, y0, onesv):
    mesh = plsc.VectorSubcoreMesh(
        core_axis_name="c", subcore_axis_name="s", num_cores=1
    )

    @functools.partial(
        pl.kernel,
        out_type=jax.ShapeDtypeStruct((_NPAD,), jnp.float32),
        mesh=mesh,
        scratch_types=[
            pltpu.VMEM((_EC,), jnp.int32),
            pltpu.VMEM((_EC,), jnp.int32),
            pltpu.VMEM((_EC,), jnp.float32),
            pltpu.VMEM((_NT,), jnp.float32),
            pltpu.VMEM((_NT,), jnp.float32),
            pltpu.VMEM((_NT,), jnp.float32),
            pltpu.VMEM((_NT,), jnp.float32),
            pltpu.VMEM((_NT,), jnp.float32),
            pltpu.VMEM_SHARED((_NPAD,), jnp.float32),
            pltpu.VMEM_SHARED((_NPAD,), jnp.float32),
            pltpu.SemaphoreType.DMA,
        ],
    )
    def run(src_hbm, dst_hbm, y0_hbm, ones_hbm, out_hbm,
            src_t, dst_t, vals, y_t, z_t, dinv_t, tmp_t, zer_t,
            acc_sh, z_sh, dma_sem):
        sid = lax.axis_index("s")
        base = sid * _NT
        ebase = sid * _EC

        cp_dst = pltpu.async_copy(dst_hbm.at[pl.ds(ebase, _EC)], dst_t, dma_sem)
        cp_ones = pltpu.async_copy(ones_hbm, vals, dma_sem)
        cp_src = pltpu.async_copy(src_hbm.at[pl.ds(ebase, _EC)], src_t, dma_sem)
        cp_y0 = pltpu.async_copy(y0_hbm.at[pl.ds(base, _NT)], y_t, dma_sem)

        def fill_zeros(i, _):
            zer_t[pl.ds(i * _VL, _VL)] = jnp.zeros((_VL,), jnp.float32)
            return 0

        lax.fori_loop(0, _NT // _VL, fill_zeros, 0)
        pltpu.sync_copy(zer_t, acc_sh.at[pl.ds(base, _NT)])
        cp_dst.wait()
        cp_ones.wait()
        plsc.subcore_barrier()

        cp_deg = pltpu.async_copy(vals, acc_sh.at[dst_t], dma_sem, add=True)
        cp_src.wait()
        cp_y0.wait()
        cp_deg.wait()
        plsc.subcore_barrier()

        pltpu.sync_copy(acc_sh.at[pl.ds(base, _NT)], tmp_t)

        def mk_dinv(i, _):
            for u in range(4):
                s = pl.ds(i * 4 * _VL + u * _VL, _VL)
                dg = tmp_t[s] + 1.0
                bits = lax.bitcast_convert_type(dg, jnp.int32)
                bits = 0x5F3759DF - lax.shift_right_arithmetic(bits, 1)
                r = lax.bitcast_convert_type(bits, jnp.float32)
                for _ in range(3):
                    r = r * (1.5 - 0.5 * dg * r * r)
                dinv_t[s] = r
            return 0

        lax.fori_loop(0, _NT // (4 * _VL), mk_dinv, 0)
        pltpu.sync_copy(zer_t, acc_sh.at[pl.ds(base, _NT)])

        for k in range(_K):
            def mk_z(i, _):
                for u in range(4):
                    s = pl.ds(i * 4 * _VL + u * _VL, _VL)
                    z_t[s] = dinv_t[s] * y_t[s]
                return 0

            lax.fori_loop(0, _NT // (4 * _VL), mk_z, 0)
            pltpu.sync_copy(z_t, z_sh.at[pl.ds(base, _NT)])
            plsc.subcore_barrier()

            pltpu.sync_copy(z_sh.at[src_t], vals)
            pltpu.sync_copy(vals, acc_sh.at[dst_t], add=True)
            plsc.subcore_barrier()

            pltpu.sync_copy(acc_sh.at[pl.ds(base, _NT)], tmp_t)
            if k < _K - 1:
                pltpu.sync_copy(zer_t, acc_sh.at[pl.ds(base, _NT)])

            def upd_y(i, _):
                for u in range(4):
                    s = pl.ds(i * 4 * _VL + u * _VL, _VL)
                    y_t[s] = dinv_t[s] * (tmp_t[s] + z_t[s])
                return 0

            lax.fori_loop(0, _NT // (4 * _VL), upd_y, 0)

        pltpu.sync_copy(y_t, out_hbm.at[pl.ds(base, _NT)])

    return run(srcv, dstv, y0, onesv)


def kernel(x, edge_index, W, b):
    ei = edge_index.astype(jnp.int32)
    y0 = _matvec(x, W).reshape(_N)
    y0p = jnp.pad(y0, (0, _NPAD - _N))
    onesv = jnp.ones((_EC,), jnp.float32)
    out = _sc_propagate(ei[0], ei[1], y0p, onesv)
    return out[:_N] + b[0]


# speedup vs baseline: 162.6699x; 1.3380x over previous
import functools

import jax
import jax.numpy as jnp
from jax import lax
from jax.experimental import pallas as pl
from jax.experimental.pallas import tpu as pltpu
from jax.experimental.pallas import tpu_sc as plsc

_N = 10000
_D = 128
_E = 320000
_K = 2

_NW = 32
_NPAD = 10240
_NT = _NPAD // 16
_EC = _E // _NW
_VL = 16


def _matvec(x, W):
    def body(x_ref, w_ref, o_ref):
        o_ref[...] = jnp.sum(x_ref[...] * w_ref[...], axis=1, keepdims=True)

    return pl.pallas_call(
        body,
        grid=(10,),
        in_specs=[
            pl.BlockSpec((_N // 10, _D), lambda i: (i, 0)),
            pl.BlockSpec((1, _D), lambda i: (0, 0)),
        ],
        out_specs=pl.BlockSpec((_N // 10, 1), lambda i: (i, 0)),
        out_shape=jax.ShapeDtypeStruct((_N, 1), jnp.float32),
    )(x, W)


def _sc_propagate(eiv, y0, onesv):
    mesh = plsc.VectorSubcoreMesh(
        core_axis_name="c", subcore_axis_name="s", num_cores=2
    )

    @functools.partial(
        pl.kernel,
        out_type=[
            jax.ShapeDtypeStruct((_NPAD,), jnp.float32),
            jax.ShapeDtypeStruct((2 * _NPAD,), jnp.float32),
        ],
        mesh=mesh,
        scratch_types=[
            pltpu.VMEM((_EC,), jnp.int32),
            pltpu.VMEM((_EC,), jnp.int32),
            pltpu.VMEM((_EC,), jnp.float32),
            pltpu.VMEM((_NT,), jnp.float32),
            pltpu.VMEM((_NT,), jnp.float32),
            pltpu.VMEM((_NT,), jnp.float32),
            pltpu.VMEM((_NT,), jnp.float32),
            pltpu.VMEM((_NT,), jnp.float32),
            pltpu.VMEM((_NT,), jnp.float32),
            pltpu.VMEM_SHARED((_NPAD,), jnp.float32),
            pltpu.VMEM_SHARED((_NPAD,), jnp.float32),
            pltpu.SemaphoreType.DMA,
            pltpu.SemaphoreType.REGULAR,
        ],
    )
    def run(ei_hbm, y0_hbm, ones_hbm,
            out_hbm, xch_hbm,
            src_t, dst_t, vals, y_t, z_t, dinv_t, tmp2_t, tmp3_t,
            zer_t, acc_sh, z_sh, dma_sem, xsem):
        cid = lax.axis_index("c")
        sid = lax.axis_index("s")
        wid = cid * 16 + sid
        nb = sid * _NT
        ebase = wid * _EC

        def xbarrier():
            plsc.subcore_barrier()
            pl.semaphore_signal(xsem, 1, core_index=1 - cid)
            pl.semaphore_wait(xsem, 1)

        def combine_and_rezero(rezero):
            pltpu.sync_copy(acc_sh.at[pl.ds(nb, _NT)], tmp3_t)
            pltpu.sync_copy(tmp3_t, xch_hbm.at[pl.ds(cid * _NPAD + nb, _NT)])
            xbarrier()
            cp_in = pltpu.async_copy(
                xch_hbm.at[pl.ds((1 - cid) * _NPAD + nb, _NT)], tmp2_t,
                dma_sem)
            if rezero:
                pltpu.sync_copy(zer_t, acc_sh.at[pl.ds(nb, _NT)])
            cp_in.wait()

        cp_dst = pltpu.async_copy(ei_hbm.at[pl.ds(_E + ebase, _EC)], dst_t,
                                  dma_sem)
        cp_ones = pltpu.async_copy(ones_hbm, vals, dma_sem)
        cp_src = pltpu.async_copy(ei_hbm.at[pl.ds(ebase, _EC)], src_t, dma_sem)
        cp_y0 = pltpu.async_copy(y0_hbm.at[pl.ds(nb, _NT)], y_t, dma_sem)

        def fill_zeros(i, _):
            for u in range(4):
                zer_t[pl.ds(i * 4 * _VL + u * _VL, _VL)] = (
                    jnp.zeros((_VL,), jnp.float32))
            return 0

        lax.fori_loop(0, _NT // (4 * _VL), fill_zeros, 0)
        pltpu.sync_copy(zer_t, acc_sh.at[pl.ds(nb, _NT)])
        cp_dst.wait()
        cp_ones.wait()
        plsc.subcore_barrier()

        cp_deg = pltpu.async_copy(vals, acc_sh.at[dst_t], dma_sem, add=True)
        cp_src.wait()
        cp_y0.wait()
        cp_deg.wait()
        plsc.subcore_barrier()

        combine_and_rezero(rezero=True)

        def mk_dinv(i, _):
            for u in range(4):
                s = pl.ds(i * 4 * _VL + u * _VL, _VL)
                dg = tmp3_t[s] + tmp2_t[s] + 1.0
                bits = lax.bitcast_convert_type(dg, jnp.int32)
                bits = 0x5F3759DF - lax.shift_right_arithmetic(bits, 1)
                r = lax.bitcast_convert_type(bits, jnp.float32)
                for _ in range(3):
                    r = r * (1.5 - 0.5 * dg * r * r)
                dinv_t[s] = r
                z_t[s] = r * y_t[s]
            return 0

        lax.fori_loop(0, _NT // (4 * _VL), mk_dinv, 0)
        pltpu.sync_copy(z_t, z_sh.at[pl.ds(nb, _NT)])
        plsc.subcore_barrier()

        for k in range(_K):
            pltpu.sync_copy(z_sh.at[src_t], vals)
            pltpu.sync_copy(vals, acc_sh.at[dst_t], add=True)
            plsc.subcore_barrier()

            combine_and_rezero(rezero=k < _K - 1)

            if k < _K - 1:
                def upd_yz(i, _):
                    for u in range(4):
                        s = pl.ds(i * 4 * _VL + u * _VL, _VL)
                        yv = dinv_t[s] * (tmp3_t[s] + tmp2_t[s] + z_t[s])
                        y_t[s] = yv
                        z_t[s] = dinv_t[s] * yv
                    return 0

                lax.fori_loop(0, _NT // (4 * _VL), upd_yz, 0)
                pltpu.sync_copy(z_t, z_sh.at[pl.ds(nb, _NT)])
                plsc.subcore_barrier()
            else:
                def upd_y(i, _):
                    for u in range(4):
                        s = pl.ds(i * 4 * _VL + u * _VL, _VL)
                        y_t[s] = dinv_t[s] * (tmp3_t[s] + tmp2_t[s] + z_t[s])
                    return 0

                lax.fori_loop(0, _NT // (4 * _VL), upd_y, 0)

                @pl.when(cid == 0)
                def _():
                    pltpu.sync_copy(y_t, out_hbm.at[pl.ds(nb, _NT)])

    return run(eiv, y0, onesv)[0]


def kernel(x, edge_index, W, b):
    eiv = edge_index.astype(jnp.int32).reshape(2 * _E)
    y0 = _matvec(x, W).reshape(_N)
    y0p = jnp.pad(y0, (0, _NPAD - _N))
    onesv = jnp.ones((_EC,), jnp.float32)
    out = _sc_propagate(eiv, y0p, onesv)
    return out[:_N] + b[0]


# speedup vs baseline: 163.5849x; 1.0056x over previous
import functools

import jax
import jax.numpy as jnp
from jax import lax
from jax.experimental import pallas as pl
from jax.experimental.pallas import tpu as pltpu
from jax.experimental.pallas import tpu_sc as plsc

_N = 10000
_D = 128
_E = 320000
_K = 2

_NW = 32
_NPAD = 10240
_NT = _NPAD // 16
_EC = _E // _NW
_VL = 16


def _matvec(x, W):
    def body(x_ref, w_ref, o_ref):
        o_ref[...] = jnp.sum(x_ref[...] * w_ref[...], axis=1, keepdims=True)

    return pl.pallas_call(
        body,
        grid=(10,),
        in_specs=[
            pl.BlockSpec((_N // 10, _D), lambda i: (i, 0)),
            pl.BlockSpec((1, _D), lambda i: (0, 0)),
        ],
        out_specs=pl.BlockSpec((_N // 10, 1), lambda i: (i, 0)),
        out_shape=jax.ShapeDtypeStruct((_N, 1), jnp.float32),
    )(x, W)


def _sc_propagate(eiv, y0, onesv):
    mesh = plsc.VectorSubcoreMesh(
        core_axis_name="c", subcore_axis_name="s", num_cores=2
    )

    @functools.partial(
        pl.kernel,
        out_type=[
            jax.ShapeDtypeStruct((_NPAD,), jnp.float32),
            jax.ShapeDtypeStruct((2 * _NPAD,), jnp.float32),
        ],
        mesh=mesh,
        scratch_types=[
            pltpu.VMEM((_EC,), jnp.int32),
            pltpu.VMEM((_EC,), jnp.int32),
            pltpu.VMEM((_EC,), jnp.float32),
            pltpu.VMEM((_NT,), jnp.float32),
            pltpu.VMEM((_NT,), jnp.float32),
            pltpu.VMEM((_NT,), jnp.float32),
            pltpu.VMEM((_NT,), jnp.float32),
            pltpu.VMEM((_NT,), jnp.float32),
            pltpu.VMEM((_NT,), jnp.float32),
            pltpu.VMEM_SHARED((_NPAD,), jnp.float32),
            pltpu.VMEM_SHARED((_NPAD,), jnp.float32),
            pltpu.SemaphoreType.DMA,
            pltpu.SemaphoreType.REGULAR,
        ],
    )
    def run(ei_hbm, y0_hbm, ones_hbm,
            out_hbm, xch_hbm,
            src_t, dst_t, vals, y_t, z_t, dinv_t, tmp2_t, tmp3_t,
            zer_t, acc_sh, z_sh, dma_sem, xsem):
        cid = lax.axis_index("c")
        sid = lax.axis_index("s")
        wid = cid * 16 + sid
        nb = sid * _NT
        ebase = wid * _EC

        def xbarrier():
            pl.semaphore_signal(xsem, 1, core_index=1 - cid)
            pl.semaphore_wait(xsem, 1)

        def combine_and_rezero(rezero):
            pltpu.sync_copy(acc_sh.at[pl.ds(nb, _NT)], tmp3_t)
            pltpu.sync_copy(tmp3_t, xch_hbm.at[pl.ds(cid * _NPAD + nb, _NT)])
            xbarrier()
            cp_in = pltpu.async_copy(
                xch_hbm.at[pl.ds((1 - cid) * _NPAD + nb, _NT)], tmp2_t,
                dma_sem)
            if rezero:
                pltpu.sync_copy(zer_t, acc_sh.at[pl.ds(nb, _NT)])
            cp_in.wait()

        cp_dst = pltpu.async_copy(ei_hbm.at[pl.ds(_E + ebase, _EC)], dst_t,
                                  dma_sem)
        cp_ones = pltpu.async_copy(ones_hbm, vals, dma_sem)
        cp_src = pltpu.async_copy(ei_hbm.at[pl.ds(ebase, _EC)], src_t, dma_sem)
        cp_y0 = pltpu.async_copy(y0_hbm.at[pl.ds(nb, _NT)], y_t, dma_sem)

        def fill_zeros(i, _):
            for u in range(4):
                zer_t[pl.ds(i * 4 * _VL + u * _VL, _VL)] = (
                    jnp.zeros((_VL,), jnp.float32))
            return 0

        lax.fori_loop(0, _NT // (4 * _VL), fill_zeros, 0)
        pltpu.sync_copy(zer_t, acc_sh.at[pl.ds(nb, _NT)])
        cp_dst.wait()
        cp_ones.wait()
        plsc.subcore_barrier()

        cp_deg = pltpu.async_copy(vals, acc_sh.at[dst_t], dma_sem, add=True)
        cp_src.wait()
        cp_y0.wait()
        cp_deg.wait()
        plsc.subcore_barrier()

        combine_and_rezero(rezero=True)

        def mk_dinv(i, _):
            for u in range(4):
                s = pl.ds(i * 4 * _VL + u * _VL, _VL)
                dg = tmp3_t[s] + tmp2_t[s] + 1.0
                bits = lax.bitcast_convert_type(dg, jnp.int32)
                bits = 0x5F3759DF - lax.shift_right_arithmetic(bits, 1)
                r = lax.bitcast_convert_type(bits, jnp.float32)
                for _ in range(3):
                    r = r * (1.5 - 0.5 * dg * r * r)
                dinv_t[s] = r
                z_t[s] = r * y_t[s]
            return 0

        lax.fori_loop(0, _NT // (4 * _VL), mk_dinv, 0)
        pltpu.sync_copy(z_t, z_sh.at[pl.ds(nb, _NT)])
        plsc.subcore_barrier()

        for k in range(_K):
            h = _EC // 2
            g0 = pltpu.async_copy(z_sh.at[src_t.at[pl.ds(0, h)]],
                                  vals.at[pl.ds(0, h)], dma_sem)
            g1 = pltpu.async_copy(z_sh.at[src_t.at[pl.ds(h, h)]],
                                  vals.at[pl.ds(h, h)], dma_sem)
            g0.wait()
            s0 = pltpu.async_copy(vals.at[pl.ds(0, h)],
                                  acc_sh.at[dst_t.at[pl.ds(0, h)]],
                                  dma_sem, add=True)
            g1.wait()
            s1 = pltpu.async_copy(vals.at[pl.ds(h, h)],
                                  acc_sh.at[dst_t.at[pl.ds(h, h)]],
                                  dma_sem, add=True)
            s0.wait()
            s1.wait()
            plsc.subcore_barrier()

            combine_and_rezero(rezero=k < _K - 1)

            if k < _K - 1:
                def upd_yz(i, _):
                    for u in range(4):
                        s = pl.ds(i * 4 * _VL + u * _VL, _VL)
                        yv = dinv_t[s] * (tmp3_t[s] + tmp2_t[s] + z_t[s])
                        y_t[s] = yv
                        z_t[s] = dinv_t[s] * yv
                    return 0

                lax.fori_loop(0, _NT // (4 * _VL), upd_yz, 0)
                pltpu.sync_copy(z_t, z_sh.at[pl.ds(nb, _NT)])
                plsc.subcore_barrier()
            else:
                def upd_y(i, _):
                    for u in range(4):
                        s = pl.ds(i * 4 * _VL + u * _VL, _VL)
                        y_t[s] = dinv_t[s] * (tmp3_t[s] + tmp2_t[s] + z_t[s])
                    return 0

                lax.fori_loop(0, _NT // (4 * _VL), upd_y, 0)

                @pl.when(cid == 0)
                def _():
                    pltpu.sync_copy(y_t, out_hbm.at[pl.ds(nb, _NT)])

    return run(eiv, y0, onesv)[0]


def kernel(x, edge_index, W, b):
    eiv = edge_index.astype(jnp.int32).reshape(2 * _E)
    y0 = _matvec(x, W).reshape(_N)
    y0p = jnp.pad(y0, (0, _NPAD - _N))
    onesv = jnp.ones((_EC,), jnp.float32)
    out = _sc_propagate(eiv, y0p, onesv)
    return out[:_N] + b[0]


# speedup vs baseline: 163.9359x; 1.0021x over previous
import functools

import jax
import jax.numpy as jnp
from jax import lax
from jax.experimental import pallas as pl
from jax.experimental.pallas import tpu as pltpu
from jax.experimental.pallas import tpu_sc as plsc

_N = 10000
_D = 128
_E = 320000
_K = 2

_NW = 32
_NPAD = 10240
_NT = _NPAD // 16
_EC = _E // _NW
_VL = 16


def _matvec(x, W):
    def body(x_ref, w_ref, o_ref):
        o_ref[...] = jnp.sum(x_ref[...] * w_ref[...], axis=1, keepdims=True)

    return pl.pallas_call(
        body,
        grid=(10,),
        in_specs=[
            pl.BlockSpec((_N // 10, _D), lambda i: (i, 0)),
            pl.BlockSpec((1, _D), lambda i: (0, 0)),
        ],
        out_specs=pl.BlockSpec((_N // 10, 1), lambda i: (i, 0)),
        out_shape=jax.ShapeDtypeStruct((_N, 1), jnp.float32),
    )(x, W)


def _sc_propagate(eiv, y0, onesv):
    mesh = plsc.VectorSubcoreMesh(
        core_axis_name="c", subcore_axis_name="s", num_cores=2
    )

    @functools.partial(
        pl.kernel,
        out_type=[
            jax.ShapeDtypeStruct((_NPAD,), jnp.float32),
            jax.ShapeDtypeStruct((2 * _NPAD,), jnp.float32),
        ],
        mesh=mesh,
        scratch_types=[
            pltpu.VMEM((_EC,), jnp.int32),
            pltpu.VMEM((_EC,), jnp.int32),
            pltpu.VMEM((_EC,), jnp.float32),
            pltpu.VMEM((_NT,), jnp.float32),
            pltpu.VMEM((_NT,), jnp.float32),
            pltpu.VMEM((_NT,), jnp.float32),
            pltpu.VMEM((_NT,), jnp.float32),
            pltpu.VMEM((_NT,), jnp.float32),
            pltpu.VMEM((_NT,), jnp.float32),
            pltpu.VMEM_SHARED((_NPAD,), jnp.float32),
            pltpu.VMEM_SHARED((_NPAD,), jnp.float32),
            pltpu.SemaphoreType.DMA,
            pltpu.SemaphoreType.REGULAR,
        ],
    )
    def run(ei_hbm, y0_hbm, ones_hbm,
            out_hbm, xch_hbm,
            src_t, dst_t, vals, y_t, z_t, dinv_t, tmp2_t, tmp3_t,
            zer_t, acc_sh, z_sh, dma_sem, xsem):
        cid = lax.axis_index("c")
        sid = lax.axis_index("s")
        wid = cid * 16 + sid
        nb = sid * _NT
        ebase = wid * _EC

        def xbarrier():
            pl.semaphore_signal(xsem, 1, core_index=1 - cid)
            pl.semaphore_wait(xsem, 1)

        def combine_and_rezero(rezero):
            pltpu.sync_copy(acc_sh.at[pl.ds(nb, _NT)], tmp3_t)
            pltpu.sync_copy(tmp3_t, xch_hbm.at[pl.ds(cid * _NPAD + nb, _NT)])
            xbarrier()
            cp_in = pltpu.async_copy(
                xch_hbm.at[pl.ds((1 - cid) * _NPAD + nb, _NT)], tmp2_t,
                dma_sem)
            if rezero:
                pltpu.sync_copy(zer_t, acc_sh.at[pl.ds(nb, _NT)])
            cp_in.wait()

        cp_dst = pltpu.async_copy(ei_hbm.at[pl.ds(_E + ebase, _EC)], dst_t,
                                  dma_sem)
        cp_ones = pltpu.async_copy(ones_hbm, vals, dma_sem)
        cp_src = pltpu.async_copy(ei_hbm.at[pl.ds(ebase, _EC)], src_t, dma_sem)
        cp_y0 = pltpu.async_copy(y0_hbm.at[pl.ds(nb, _NT)], y_t, dma_sem)

        def fill_zeros(i, _):
            for u in range(4):
                zer_t[pl.ds(i * 4 * _VL + u * _VL, _VL)] = (
                    jnp.zeros((_VL,), jnp.float32))
            return 0

        lax.fori_loop(0, _NT // (4 * _VL), fill_zeros, 0)
        pltpu.sync_copy(zer_t, acc_sh.at[pl.ds(nb, _NT)])
        cp_dst.wait()
        cp_ones.wait()
        plsc.subcore_barrier()

        cp_deg = pltpu.async_copy(vals, acc_sh.at[dst_t], dma_sem, add=True)
        cp_src.wait()
        cp_y0.wait()
        cp_deg.wait()
        plsc.subcore_barrier()

        combine_and_rezero(rezero=True)

        def mk_dinv(i, _):
            for u in range(4):
                s = pl.ds(i * 4 * _VL + u * _VL, _VL)
                dg = tmp3_t[s] + tmp2_t[s] + 1.0
                bits = lax.bitcast_convert_type(dg, jnp.int32)
                bits = 0x5F3759DF - lax.shift_right_arithmetic(bits, 1)
                r = lax.bitcast_convert_type(bits, jnp.float32)
                for _ in range(3):
                    r = r * (1.5 - 0.5 * dg * r * r)
                dinv_t[s] = r
                z_t[s] = r * y_t[s]
            return 0

        lax.fori_loop(0, _NT // (4 * _VL), mk_dinv, 0)
        pltpu.sync_copy(z_t, z_sh.at[pl.ds(nb, _NT)])
        plsc.subcore_barrier()

        for k in range(_K):
            h = _EC // 2
            g0 = pltpu.async_copy(z_sh.at[src_t.at[pl.ds(0, h)]],
                                  vals.at[pl.ds(0, h)], dma_sem)
            g1 = pltpu.async_copy(z_sh.at[src_t.at[pl.ds(h, h)]],
                                  vals.at[pl.ds(h, h)], dma_sem)
            g0.wait()
            s0 = pltpu.async_copy(vals.at[pl.ds(0, h)],
                                  acc_sh.at[dst_t.at[pl.ds(0, h)]],
                                  dma_sem, add=True)
            g1.wait()
            s1 = pltpu.async_copy(vals.at[pl.ds(h, h)],
                                  acc_sh.at[dst_t.at[pl.ds(h, h)]],
                                  dma_sem, add=True)
            s0.wait()
            s1.wait()
            plsc.subcore_barrier()

            if k < _K - 1:
                combine_and_rezero(rezero=True)
                def upd_yz(i, _):
                    for u in range(4):
                        s = pl.ds(i * 4 * _VL + u * _VL, _VL)
                        yv = dinv_t[s] * (tmp3_t[s] + tmp2_t[s] + z_t[s])
                        y_t[s] = yv
                        z_t[s] = dinv_t[s] * yv
                    return 0

                lax.fori_loop(0, _NT // (4 * _VL), upd_yz, 0)
                pltpu.sync_copy(z_t, z_sh.at[pl.ds(nb, _NT)])
                plsc.subcore_barrier()
            else:
                @pl.when(cid == 1)
                def _():
                    pltpu.sync_copy(acc_sh.at[pl.ds(nb, _NT)], tmp3_t)
                    pltpu.sync_copy(tmp3_t,
                                    xch_hbm.at[pl.ds(_NPAD + nb, _NT)])
                xbarrier()

                @pl.when(cid == 0)
                def _():
                    cp_in = pltpu.async_copy(
                        xch_hbm.at[pl.ds(_NPAD + nb, _NT)], tmp2_t, dma_sem)
                    pltpu.sync_copy(acc_sh.at[pl.ds(nb, _NT)], tmp3_t)
                    cp_in.wait()

                def upd_y(i, _):
                    for u in range(4):
                        s = pl.ds(i * 4 * _VL + u * _VL, _VL)
                        y_t[s] = dinv_t[s] * (tmp3_t[s] + tmp2_t[s] + z_t[s])
                    return 0

                lax.fori_loop(0, _NT // (4 * _VL), upd_y, 0)

                @pl.when(cid == 0)
                def _():
                    pltpu.sync_copy(y_t, out_hbm.at[pl.ds(nb, _NT)])

    return run(eiv, y0, onesv)[0]


def kernel(x, edge_index, W, b):
    eiv = edge_index.astype(jnp.int32).reshape(2 * _E)
    y0 = _matvec(x, W).reshape(_N)
    y0p = jnp.pad(y0, (0, _NPAD - _N))
    onesv = jnp.ones((_EC,), jnp.float32)
    out = _sc_propagate(eiv, y0p, onesv)
    return out[:_N] + b[0]
